# Initial kernel scaffold; baseline (speedup 1.0000x reference)
import jax
import jax.numpy as jnp
from jax.experimental import pallas as pl


def kernel(x, edge_index, Ws1_0, Wn1_0, b1_0, Ws2_0, Wn2_0, b2_0, Ws1_1, Wn1_1, b1_1, Ws2_1, Wn2_1, b2_1):
    raise NotImplementedError("write your pallas kernel here")


# speedup vs baseline: 2.5466x; 2.5466x over previous
import functools

import jax
import jax.numpy as jnp
from jax import lax
from jax.experimental import pallas as pl
from jax.experimental.pallas import tpu as pltpu
from jax.experimental.pallas import tpu_sc as plsc

N = 10000
D = 128
E = 320000

NC = 2
NS = 16
NW = NC * NS
E_PAD = 327680
EPW = E_PAD // NW
CHUNK = 128
STEPS = EPW // CHUNK
N_PAD = 10240
RPT = N_PAD // NS
JUNK_ROW = N


def _spmm_body(x_hbm, src_hbm, dst_hbm, zeros_hbm, out_hbm,
               src_v, dst_v, rows_v, acc_sh, sem):
    c = lax.axis_index("c")
    s = lax.axis_index("s")
    wid = s * NC + c
    r0 = s * RPT
    pltpu.sync_copy(zeros_hbm.at[pl.ds(r0, RPT)], acc_sh.at[pl.ds(r0, RPT)])
    plsc.subcore_barrier()

    ebase = wid * EPW

    def step(i, carry):
        b = ebase + i * CHUNK
        pltpu.sync_copy(src_hbm.at[pl.ds(b, CHUNK)], src_v)
        pltpu.sync_copy(dst_hbm.at[pl.ds(b, CHUNK)], dst_v)
        pltpu.async_copy(x_hbm.at[src_v], rows_v, sem).wait()
        pltpu.sync_copy(rows_v, acc_sh.at[dst_v], add=True)
        return carry

    lax.fori_loop(0, STEPS, step, 0)
    plsc.subcore_barrier()
    pltpu.sync_copy(acc_sh.at[pl.ds(r0, RPT)], out_hbm.at[c, pl.ds(r0, RPT)])


def _make_spmm():
    mesh = plsc.VectorSubcoreMesh(core_axis_name="c", subcore_axis_name="s")
    return functools.partial(
        pl.kernel,
        mesh=mesh,
        out_type=jax.ShapeDtypeStruct((NC, N_PAD, D), jnp.float32),
        scratch_types=[
            pltpu.VMEM((CHUNK,), jnp.int32),
            pltpu.VMEM((CHUNK,), jnp.int32),
            pltpu.VMEM((CHUNK, D), jnp.float32),
            pltpu.VMEM_SHARED((N_PAD, D), jnp.float32),
            pltpu.SemaphoreType.DMA,
        ],
    )(_spmm_body)


_spmm = _make_spmm()


_R = 2000
_GRID = N // _R


def _conv1_body(x_ref, p_ref, ws_ref, wn_ref, b_ref, o_ref):
    agg = p_ref[0] + p_ref[1]
    acc = jnp.dot(x_ref[...], ws_ref[...], preferred_element_type=jnp.float32)
    acc = acc + jnp.dot(agg, wn_ref[...], preferred_element_type=jnp.float32)
    acc = acc + b_ref[...]
    o_ref[...] = jnp.maximum(acc, 0.0)


def _conv2_body(h_ref, p_ref, res_ref, ws_ref, wn_ref, b_ref, o_ref):
    agg = p_ref[0] + p_ref[1]
    acc = jnp.dot(h_ref[...], ws_ref[...], preferred_element_type=jnp.float32)
    acc = acc + jnp.dot(agg, wn_ref[...], preferred_element_type=jnp.float32)
    acc = acc + b_ref[...] + res_ref[...]
    o_ref[...] = jnp.maximum(acc, 0.0)


_row_spec = pl.BlockSpec((_R, D), lambda i: (i, 0))
_p_spec = pl.BlockSpec((NC, _R, D), lambda i: (0, i, 0))
_w_spec = pl.BlockSpec((D, D), lambda i: (0, 0))
_b_spec = pl.BlockSpec((1, D), lambda i: (0, 0))
_out_shape = jax.ShapeDtypeStruct((N, D), jnp.float32)

_conv1 = pl.pallas_call(
    _conv1_body,
    grid=(_GRID,),
    in_specs=[_row_spec, _p_spec, _w_spec, _w_spec, _b_spec],
    out_specs=_row_spec,
    out_shape=_out_shape,
)

_conv2 = pl.pallas_call(
    _conv2_body,
    grid=(_GRID,),
    in_specs=[_row_spec, _p_spec, _row_spec, _w_spec, _w_spec, _b_spec],
    out_specs=_row_spec,
    out_shape=_out_shape,
)


def kernel(x, edge_index, Ws1_0, Wn1_0, b1_0, Ws2_0, Wn2_0, b2_0,
           Ws1_1, Wn1_1, b1_1, Ws2_1, Wn2_1, b2_1):
    src = edge_index[0]
    dst = edge_index[1]
    pad = E_PAD - E
    src_p = jnp.concatenate([src, jnp.zeros((pad,), jnp.int32)])
    dst_p = jnp.concatenate([dst, jnp.full((pad,), JUNK_ROW, jnp.int32)])
    zeros = jnp.zeros((N_PAD, D), jnp.float32)

    def block(x, Ws1, Wn1, b1, Ws2, Wn2, b2):
        p1 = _spmm(x, src_p, dst_p, zeros)
        h = _conv1(x, p1, Ws1, Wn1, b1.reshape(1, D))
        p2 = _spmm(h, src_p, dst_p, zeros)
        return _conv2(h, p2, x, Ws2, Wn2, b2.reshape(1, D))

    x = block(x, Ws1_0, Wn1_0, b1_0, Ws2_0, Wn2_0, b2_0)
    x = block(x, Ws1_1, Wn1_1, b1_1, Ws2_1, Wn2_1, b2_1)
    return (x, edge_index)


# speedup vs baseline: 3.2544x; 1.2779x over previous
import functools

import jax
import jax.numpy as jnp
from jax import lax
from jax.experimental import pallas as pl
from jax.experimental.pallas import tpu as pltpu
from jax.experimental.pallas import tpu_sc as plsc

N = 10000
D = 128
E = 320000

NC = 2
NS = 16
NW = NC * NS
E_PAD = 327680
EPW = E_PAD // NW
CHUNK = 128
STEPS = EPW // CHUNK
N_PAD = 10112
RPT = N_PAD // NS
JUNK_ROW = N


def _spmm_body(x_hbm, src_hbm, dst_hbm, zeros_hbm, out_hbm,
               src_v, dst0, dst1, rows0, rows1, acc_sh, g0, g1, d0, d1):
    c = lax.axis_index("c")
    s = lax.axis_index("s")
    wid = s * NC + c
    r0 = s * RPT
    pltpu.sync_copy(zeros_hbm.at[pl.ds(r0, RPT)], acc_sh.at[pl.ds(r0, RPT)])
    ib = wid * STEPS
    pltpu.sync_copy(src_hbm.at[pl.ds(ib, STEPS)], src_v)
    plsc.subcore_barrier()

    pltpu.async_copy(dst_hbm.at[ib], dst0, d0)
    pltpu.async_copy(x_hbm.at[src_v.at[0]], rows0, g0)

    def step(i, carry):
        even = lax.rem(i, 2) == 0
        more = i + 1 < STEPS

        @pl.when(jnp.logical_and(even, more))
        def _():
            pltpu.async_copy(dst_hbm.at[ib + i + 1], dst1, d1)
            pltpu.async_copy(x_hbm.at[src_v.at[i + 1]], rows1, g1)

        @pl.when(jnp.logical_and(jnp.logical_not(even), more))
        def _():
            pltpu.async_copy(dst_hbm.at[ib + i + 1], dst0, d0)
            pltpu.async_copy(x_hbm.at[src_v.at[i + 1]], rows0, g0)

        @pl.when(even)
        def _():
            pltpu.make_async_copy(dst_hbm.at[ib + i], dst0, d0).wait()
            pltpu.make_async_copy(x_hbm.at[src_v.at[i]], rows0, g0).wait()
            pltpu.sync_copy(rows0, acc_sh.at[dst0], add=True)

        @pl.when(jnp.logical_not(even))
        def _():
            pltpu.make_async_copy(dst_hbm.at[ib + i], dst1, d1).wait()
            pltpu.make_async_copy(x_hbm.at[src_v.at[i]], rows1, g1).wait()
            pltpu.sync_copy(rows1, acc_sh.at[dst1], add=True)

        return carry

    lax.fori_loop(0, STEPS, step, 0)
    plsc.subcore_barrier()
    pltpu.sync_copy(acc_sh.at[pl.ds(r0, RPT)], out_hbm.at[c, pl.ds(r0, RPT)])


def _make_spmm():
    mesh = plsc.VectorSubcoreMesh(core_axis_name="c", subcore_axis_name="s")
    return functools.partial(
        pl.kernel,
        mesh=mesh,
        out_type=jax.ShapeDtypeStruct((NC, N_PAD, D), jnp.float32),
        scratch_types=[
            pltpu.VMEM((STEPS, CHUNK), jnp.int32),
            pltpu.VMEM((CHUNK,), jnp.int32),
            pltpu.VMEM((CHUNK,), jnp.int32),
            pltpu.VMEM((CHUNK, D), jnp.float32),
            pltpu.VMEM((CHUNK, D), jnp.float32),
            pltpu.VMEM_SHARED((N_PAD, D), jnp.float32),
            pltpu.SemaphoreType.DMA,
            pltpu.SemaphoreType.DMA,
            pltpu.SemaphoreType.DMA,
            pltpu.SemaphoreType.DMA,
        ],
    )(_spmm_body)


_spmm = _make_spmm()


_R = 2000
_GRID = N // _R


def _conv1_body(x_ref, p_ref, ws_ref, wn_ref, b_ref, o_ref):
    agg = p_ref[0] + p_ref[1]
    acc = jnp.dot(x_ref[...], ws_ref[...], preferred_element_type=jnp.float32)
    acc = acc + jnp.dot(agg, wn_ref[...], preferred_element_type=jnp.float32)
    acc = acc + b_ref[...]
    o_ref[...] = jnp.maximum(acc, 0.0)


def _conv2_body(h_ref, p_ref, res_ref, ws_ref, wn_ref, b_ref, o_ref):
    agg = p_ref[0] + p_ref[1]
    acc = jnp.dot(h_ref[...], ws_ref[...], preferred_element_type=jnp.float32)
    acc = acc + jnp.dot(agg, wn_ref[...], preferred_element_type=jnp.float32)
    acc = acc + b_ref[...] + res_ref[...]
    o_ref[...] = jnp.maximum(acc, 0.0)


_row_spec = pl.BlockSpec((_R, D), lambda i: (i, 0))
_p_spec = pl.BlockSpec((NC, _R, D), lambda i: (0, i, 0))
_w_spec = pl.BlockSpec((D, D), lambda i: (0, 0))
_b_spec = pl.BlockSpec((1, D), lambda i: (0, 0))
_out_shape = jax.ShapeDtypeStruct((N, D), jnp.float32)

_conv1 = pl.pallas_call(
    _conv1_body,
    grid=(_GRID,),
    in_specs=[_row_spec, _p_spec, _w_spec, _w_spec, _b_spec],
    out_specs=_row_spec,
    out_shape=_out_shape,
)

_conv2 = pl.pallas_call(
    _conv2_body,
    grid=(_GRID,),
    in_specs=[_row_spec, _p_spec, _row_spec, _w_spec, _w_spec, _b_spec],
    out_specs=_row_spec,
    out_shape=_out_shape,
)


def kernel(x, edge_index, Ws1_0, Wn1_0, b1_0, Ws2_0, Wn2_0, b2_0,
           Ws1_1, Wn1_1, b1_1, Ws2_1, Wn2_1, b2_1):
    src = edge_index[0]
    dst = edge_index[1]
    pad = E_PAD - E
    src_p = jnp.concatenate([src, jnp.zeros((pad,), jnp.int32)]).reshape(-1, CHUNK)
    dst_p = jnp.concatenate([dst, jnp.full((pad,), JUNK_ROW, jnp.int32)]).reshape(-1, CHUNK)
    zeros = jnp.zeros((N_PAD, D), jnp.float32)

    def block(x, Ws1, Wn1, b1, Ws2, Wn2, b2):
        p1 = _spmm(x, src_p, dst_p, zeros)
        h = _conv1(x, p1, Ws1, Wn1, b1.reshape(1, D))
        p2 = _spmm(h, src_p, dst_p, zeros)
        return _conv2(h, p2, x, Ws2, Wn2, b2.reshape(1, D))

    x = block(x, Ws1_0, Wn1_0, b1_0, Ws2_0, Wn2_0, b2_0)
    x = block(x, Ws1_1, Wn1_1, b1_1, Ws2_1, Wn2_1, b2_1)
    return (x, edge_index)


# speedup vs baseline: 12.5553x; 3.8579x over previous
import functools

import jax
import jax.numpy as jnp
from jax import lax
from jax.experimental import pallas as pl
from jax.experimental.pallas import tpu as pltpu
from jax.experimental.pallas import tpu_sc as plsc

N = 10000
D = 128
E = 320000

NC = 2
NS = 16
NW = NC * NS
E_PAD = 327680
EPW = E_PAD // NW
CHUNK = 128
STEPS = EPW // CHUNK
N_PAD = 10112
RPT = N_PAD // NS
JUNK_ROW = N


def _spmm_body(x_hbm, src_hbm, dst_hbm, zeros_hbm, out_hbm,
               src_v, dst0, dst1, rows0, rows1, acc_sh, g0, g1, d0, d1):
    c = lax.axis_index("c")
    s = lax.axis_index("s")
    wid = s * NC + c
    r0 = s * RPT
    pltpu.sync_copy(zeros_hbm.at[pl.ds(r0, RPT)], acc_sh.at[pl.ds(r0, RPT)])
    ib = wid * STEPS
    pltpu.sync_copy(src_hbm.at[pl.ds(ib, STEPS)], src_v)
    plsc.subcore_barrier()

    pltpu.async_copy(dst_hbm.at[ib], dst0, d0)
    pltpu.async_copy(x_hbm.at[src_v.at[0]], rows0, g0)

    def step(i, carry):
        even = lax.rem(i, 2) == 0
        more = i + 1 < STEPS

        @pl.when(jnp.logical_and(even, more))
        def _():
            pltpu.async_copy(dst_hbm.at[ib + i + 1], dst1, d1)
            pltpu.async_copy(x_hbm.at[src_v.at[i + 1]], rows1, g1)

        @pl.when(jnp.logical_and(jnp.logical_not(even), more))
        def _():
            pltpu.async_copy(dst_hbm.at[ib + i + 1], dst0, d0)
            pltpu.async_copy(x_hbm.at[src_v.at[i + 1]], rows0, g0)

        @pl.when(even)
        def _():
            pltpu.make_async_copy(dst_hbm.at[ib + i], dst0, d0).wait()
            pltpu.make_async_copy(x_hbm.at[src_v.at[i]], rows0, g0).wait()
            pltpu.sync_copy(rows0, acc_sh.at[dst0], add=True)

        @pl.when(jnp.logical_not(even))
        def _():
            pltpu.make_async_copy(dst_hbm.at[ib + i], dst1, d1).wait()
            pltpu.make_async_copy(x_hbm.at[src_v.at[i]], rows1, g1).wait()
            pltpu.sync_copy(rows1, acc_sh.at[dst1], add=True)

        return carry

    lax.fori_loop(0, STEPS, step, 0)
    plsc.subcore_barrier()
    pltpu.sync_copy(acc_sh.at[pl.ds(r0, RPT)], out_hbm.at[c, pl.ds(r0, RPT)])


def _make_spmm():
    mesh = plsc.VectorSubcoreMesh(core_axis_name="c", subcore_axis_name="s")
    return functools.partial(
        pl.kernel,
        mesh=mesh,
        out_type=jax.ShapeDtypeStruct((NC, N_PAD, D), jnp.float32),
        scratch_types=[
            pltpu.VMEM((STEPS, CHUNK), jnp.int32),
            pltpu.VMEM((CHUNK,), jnp.int32),
            pltpu.VMEM((CHUNK,), jnp.int32),
            pltpu.VMEM((CHUNK, D), jnp.float32),
            pltpu.VMEM((CHUNK, D), jnp.float32),
            pltpu.VMEM_SHARED((N_PAD, D), jnp.float32),
            pltpu.SemaphoreType.DMA,
            pltpu.SemaphoreType.DMA,
            pltpu.SemaphoreType.DMA,
            pltpu.SemaphoreType.DMA,
        ],
    )(_spmm_body)


_spmm = _make_spmm()


_R = 2000
_GRID = N // _R


def _conv1_body(x_ref, p_ref, ws_ref, wn_ref, b_ref, o_ref):
    agg = p_ref[0] + p_ref[1]
    acc = jnp.dot(x_ref[...], ws_ref[...], preferred_element_type=jnp.float32)
    acc = acc + jnp.dot(agg, wn_ref[...], preferred_element_type=jnp.float32)
    acc = acc + b_ref[...]
    o_ref[...] = jnp.maximum(acc, 0.0)


def _conv2_body(h_ref, p_ref, res_ref, ws_ref, wn_ref, b_ref, o_ref):
    agg = p_ref[0] + p_ref[1]
    acc = jnp.dot(h_ref[...], ws_ref[...], preferred_element_type=jnp.float32)
    acc = acc + jnp.dot(agg, wn_ref[...], preferred_element_type=jnp.float32)
    acc = acc + b_ref[...] + res_ref[...]
    o_ref[...] = jnp.maximum(acc, 0.0)


_row_spec = pl.BlockSpec((_R, D), lambda i: (i, 0))
_p_spec = pl.BlockSpec((NC, _R, D), lambda i: (0, i, 0))
_w_spec = pl.BlockSpec((D, D), lambda i: (0, 0))
_b_spec = pl.BlockSpec((1, D), lambda i: (0, 0))
_out_shape = jax.ShapeDtypeStruct((N, D), jnp.float32)

_conv1 = pl.pallas_call(
    _conv1_body,
    grid=(_GRID,),
    in_specs=[_row_spec, _p_spec, _w_spec, _w_spec, _b_spec],
    out_specs=_row_spec,
    out_shape=_out_shape,
)

_conv2 = pl.pallas_call(
    _conv2_body,
    grid=(_GRID,),
    in_specs=[_row_spec, _p_spec, _row_spec, _w_spec, _w_spec, _b_spec],
    out_specs=_row_spec,
    out_shape=_out_shape,
)


def kernel(x, edge_index, Ws1_0, Wn1_0, b1_0, Ws2_0, Wn2_0, b2_0,
           Ws1_1, Wn1_1, b1_1, Ws2_1, Wn2_1, b2_1):
    src = edge_index[0]
    dst = edge_index[1]
    pad = E_PAD - E
    k = jnp.arange(pad, dtype=jnp.int32)
    src_p = jnp.concatenate([src, k % N]).reshape(-1, CHUNK)
    dst_p = jnp.concatenate([dst, JUNK_ROW + k % (N_PAD - N)]).reshape(-1, CHUNK)
    zeros = jnp.zeros((N_PAD, D), jnp.float32)

    def block(x, Ws1, Wn1, b1, Ws2, Wn2, b2):
        p1 = _spmm(x, src_p, dst_p, zeros)
        h = _conv1(x, p1, Ws1, Wn1, b1.reshape(1, D))
        p2 = _spmm(h, src_p, dst_p, zeros)
        return _conv2(h, p2, x, Ws2, Wn2, b2.reshape(1, D))

    x = block(x, Ws1_0, Wn1_0, b1_0, Ws2_0, Wn2_0, b2_0)
    x = block(x, Ws1_1, Wn1_1, b1_1, Ws2_1, Wn2_1, b2_1)
    return (x, edge_index)


# speedup vs baseline: 12.5855x; 1.0024x over previous
import functools

import jax
import jax.numpy as jnp
from jax import lax
from jax.experimental import pallas as pl
from jax.experimental.pallas import tpu as pltpu
from jax.experimental.pallas import tpu_sc as plsc

N = 10000
D = 128
E = 320000

NC = 2
NS = 16
NW = NC * NS
E_PAD = 327680
EPW = E_PAD // NW
CHUNK = 128
STEPS = EPW // CHUNK
N_PAD = 10112
RPT = N_PAD // NS
JUNK_ROW = N


def _spmm_body(x_hbm, src_hbm, dst_hbm, zeros_hbm, out_hbm,
               src_v, dst0, dst1, rows0, rows1, acc_sh, g0, g1, d0, d1, s0, s1):
    c = lax.axis_index("c")
    s = lax.axis_index("s")
    wid = s * NC + c
    r0 = s * RPT
    pltpu.sync_copy(zeros_hbm.at[pl.ds(r0, RPT)], acc_sh.at[pl.ds(r0, RPT)])
    ib = wid * STEPS
    pltpu.sync_copy(src_hbm.at[pl.ds(ib, STEPS)], src_v)
    plsc.subcore_barrier()

    pltpu.async_copy(dst_hbm.at[ib], dst0, d0)
    pltpu.async_copy(x_hbm.at[src_v.at[0]], rows0, g0)

    def step(i, carry):
        even = lax.rem(i, 2) == 0
        more = i + 1 < STEPS

        @pl.when(even)
        def _():
            @pl.when(i >= 1)
            def _():
                pltpu.make_async_copy(rows1, acc_sh.at[dst1], s1).wait()

            @pl.when(more)
            def _():
                pltpu.async_copy(dst_hbm.at[ib + i + 1], dst1, d1)
                pltpu.async_copy(x_hbm.at[src_v.at[i + 1]], rows1, g1)

            pltpu.make_async_copy(dst_hbm.at[ib + i], dst0, d0).wait()
            pltpu.make_async_copy(x_hbm.at[src_v.at[i]], rows0, g0).wait()
            pltpu.async_copy(rows0, acc_sh.at[dst0], s0, add=True)

        @pl.when(jnp.logical_not(even))
        def _():
            pltpu.make_async_copy(rows0, acc_sh.at[dst0], s0).wait()

            @pl.when(more)
            def _():
                pltpu.async_copy(dst_hbm.at[ib + i + 1], dst0, d0)
                pltpu.async_copy(x_hbm.at[src_v.at[i + 1]], rows0, g0)

            pltpu.make_async_copy(dst_hbm.at[ib + i], dst1, d1).wait()
            pltpu.make_async_copy(x_hbm.at[src_v.at[i]], rows1, g1).wait()
            pltpu.async_copy(rows1, acc_sh.at[dst1], s1, add=True)

        return carry

    lax.fori_loop(0, STEPS, step, 0)
    pltpu.make_async_copy(rows1, acc_sh.at[dst1], s1).wait()
    plsc.subcore_barrier()
    pltpu.sync_copy(acc_sh.at[pl.ds(r0, RPT)], out_hbm.at[c, pl.ds(r0, RPT)])


def _make_spmm():
    mesh = plsc.VectorSubcoreMesh(core_axis_name="c", subcore_axis_name="s")
    return functools.partial(
        pl.kernel,
        mesh=mesh,
        out_type=jax.ShapeDtypeStruct((NC, N_PAD, D), jnp.float32),
        scratch_types=[
            pltpu.VMEM((STEPS, CHUNK), jnp.int32),
            pltpu.VMEM((CHUNK,), jnp.int32),
            pltpu.VMEM((CHUNK,), jnp.int32),
            pltpu.VMEM((CHUNK, D), jnp.float32),
            pltpu.VMEM((CHUNK, D), jnp.float32),
            pltpu.VMEM_SHARED((N_PAD, D), jnp.float32),
            pltpu.SemaphoreType.DMA,
            pltpu.SemaphoreType.DMA,
            pltpu.SemaphoreType.DMA,
            pltpu.SemaphoreType.DMA,
            pltpu.SemaphoreType.DMA,
            pltpu.SemaphoreType.DMA,
        ],
    )(_spmm_body)


_spmm = _make_spmm()


_R = 2000
_GRID = N // _R


def _conv1_body(x_ref, p_ref, ws_ref, wn_ref, b_ref, o_ref):
    agg = p_ref[0] + p_ref[1]
    acc = jnp.dot(x_ref[...], ws_ref[...], preferred_element_type=jnp.float32)
    acc = acc + jnp.dot(agg, wn_ref[...], preferred_element_type=jnp.float32)
    acc = acc + b_ref[...]
    o_ref[...] = jnp.maximum(acc, 0.0)


def _conv2_body(h_ref, p_ref, res_ref, ws_ref, wn_ref, b_ref, o_ref):
    agg = p_ref[0] + p_ref[1]
    acc = jnp.dot(h_ref[...], ws_ref[...], preferred_element_type=jnp.float32)
    acc = acc + jnp.dot(agg, wn_ref[...], preferred_element_type=jnp.float32)
    acc = acc + b_ref[...] + res_ref[...]
    o_ref[...] = jnp.maximum(acc, 0.0)


_row_spec = pl.BlockSpec((_R, D), lambda i: (i, 0))
_p_spec = pl.BlockSpec((NC, _R, D), lambda i: (0, i, 0))
_w_spec = pl.BlockSpec((D, D), lambda i: (0, 0))
_b_spec = pl.BlockSpec((1, D), lambda i: (0, 0))
_out_shape = jax.ShapeDtypeStruct((N, D), jnp.float32)

_conv1 = pl.pallas_call(
    _conv1_body,
    grid=(_GRID,),
    in_specs=[_row_spec, _p_spec, _w_spec, _w_spec, _b_spec],
    out_specs=_row_spec,
    out_shape=_out_shape,
)

_conv2 = pl.pallas_call(
    _conv2_body,
    grid=(_GRID,),
    in_specs=[_row_spec, _p_spec, _row_spec, _w_spec, _w_spec, _b_spec],
    out_specs=_row_spec,
    out_shape=_out_shape,
)


def kernel(x, edge_index, Ws1_0, Wn1_0, b1_0, Ws2_0, Wn2_0, b2_0,
           Ws1_1, Wn1_1, b1_1, Ws2_1, Wn2_1, b2_1):
    src = edge_index[0]
    dst = edge_index[1]
    pad = E_PAD - E
    k = jnp.arange(pad, dtype=jnp.int32)
    src_p = jnp.concatenate([src, k % N]).reshape(-1, CHUNK)
    dst_p = jnp.concatenate([dst, JUNK_ROW + k % (N_PAD - N)]).reshape(-1, CHUNK)
    zeros = jnp.zeros((N_PAD, D), jnp.float32)

    def block(x, Ws1, Wn1, b1, Ws2, Wn2, b2):
        p1 = _spmm(x, src_p, dst_p, zeros)
        h = _conv1(x, p1, Ws1, Wn1, b1.reshape(1, D))
        p2 = _spmm(h, src_p, dst_p, zeros)
        return _conv2(h, p2, x, Ws2, Wn2, b2.reshape(1, D))

    x = block(x, Ws1_0, Wn1_0, b1_0, Ws2_0, Wn2_0, b2_0)
    x = block(x, Ws1_1, Wn1_1, b1_1, Ws2_1, Wn2_1, b2_1)
    return (x, edge_index)


# speedup vs baseline: 12.7549x; 1.0135x over previous
import functools

import jax
import jax.numpy as jnp
from jax import lax
from jax.experimental import pallas as pl
from jax.experimental.pallas import tpu as pltpu
from jax.experimental.pallas import tpu_sc as plsc

N = 10000
D = 128
E = 320000

NC = 2
NS = 16
NW = NC * NS
EPW = E // NW
CHUNK = 128
STEPS = EPW // CHUNK
TAIL = EPW - STEPS * CHUNK
N_PAD = 10112
RPT = N_PAD // NS


def _spmm_body(x_hbm, src_hbm, dst_hbm, zeros_hbm, out_hbm,
               src_v, dst0, dst1, dstt, rows0, rows1, rowst,
               acc_sh, g0, g1, d0, d1, s0, s1):
    c = lax.axis_index("c")
    s = lax.axis_index("s")
    wid = s * NC + c
    r0 = s * RPT
    eb = wid * EPW
    pltpu.sync_copy(src_hbm.at[pl.ds(eb, EPW)], src_v)
    pltpu.async_copy(dst_hbm.at[pl.ds(eb, CHUNK)], dst0, d0)
    pltpu.async_copy(x_hbm.at[src_v.at[pl.ds(0, CHUNK)]], rows0, g0)
    pltpu.sync_copy(zeros_hbm.at[pl.ds(r0, RPT)], acc_sh.at[pl.ds(r0, RPT)])
    plsc.subcore_barrier()

    def halfstep(i, rows, dst, gsem, dsem, ssem,
                 orows, odst, ogsem, odsem, ossem):
        @pl.when(i >= 1)
        def _():
            pltpu.make_async_copy(orows, acc_sh.at[odst], ossem).wait()

        @pl.when(i + 1 < STEPS)
        def _():
            pltpu.async_copy(
                dst_hbm.at[pl.ds(eb + (i + 1) * CHUNK, CHUNK)], odst, odsem)
            pltpu.async_copy(
                x_hbm.at[src_v.at[pl.ds((i + 1) * CHUNK, CHUNK)]], orows, ogsem)

        pltpu.make_async_copy(
            dst_hbm.at[pl.ds(eb + i * CHUNK, CHUNK)], dst, dsem).wait()
        pltpu.make_async_copy(
            x_hbm.at[src_v.at[pl.ds(i * CHUNK, CHUNK)]], rows, gsem).wait()
        pltpu.async_copy(rows, acc_sh.at[dst], ssem, add=True)

    def step(i, carry):
        @pl.when(lax.rem(i, 2) == 0)
        def _():
            halfstep(i, rows0, dst0, g0, d0, s0, rows1, dst1, g1, d1, s1)

        @pl.when(lax.rem(i, 2) == 1)
        def _():
            halfstep(i, rows1, dst1, g1, d1, s1, rows0, dst0, g0, d0, s0)

        return carry

    lax.fori_loop(0, STEPS, step, 0)
    pltpu.make_async_copy(rows1, acc_sh.at[dst1], s1).wait()
    tb = eb + STEPS * CHUNK
    pltpu.sync_copy(dst_hbm.at[pl.ds(tb, TAIL)], dstt)
    pltpu.async_copy(x_hbm.at[src_v.at[pl.ds(STEPS * CHUNK, TAIL)]], rowst, g0)
    pltpu.make_async_copy(
        x_hbm.at[src_v.at[pl.ds(STEPS * CHUNK, TAIL)]], rowst, g0).wait()
    pltpu.sync_copy(rowst, acc_sh.at[dstt], add=True)
    plsc.subcore_barrier()
    pltpu.sync_copy(acc_sh.at[pl.ds(r0, RPT)], out_hbm.at[c, pl.ds(r0, RPT)])


def _make_spmm():
    mesh = plsc.VectorSubcoreMesh(core_axis_name="c", subcore_axis_name="s")
    return functools.partial(
        pl.kernel,
        mesh=mesh,
        out_type=jax.ShapeDtypeStruct((NC, N_PAD, D), jnp.float32),
        scratch_types=[
            pltpu.VMEM((EPW,), jnp.int32),
            pltpu.VMEM((CHUNK,), jnp.int32),
            pltpu.VMEM((CHUNK,), jnp.int32),
            pltpu.VMEM((TAIL,), jnp.int32),
            pltpu.VMEM((CHUNK, D), jnp.float32),
            pltpu.VMEM((CHUNK, D), jnp.float32),
            pltpu.VMEM((TAIL, D), jnp.float32),
            pltpu.VMEM_SHARED((N_PAD, D), jnp.float32),
            pltpu.SemaphoreType.DMA,
            pltpu.SemaphoreType.DMA,
            pltpu.SemaphoreType.DMA,
            pltpu.SemaphoreType.DMA,
            pltpu.SemaphoreType.DMA,
            pltpu.SemaphoreType.DMA,
        ],
    )(_spmm_body)


_spmm = _make_spmm()


_R = 2000
_GRID = N // _R


def _conva_body(x_ref, ws_ref, b_ref, o_ref):
    o_ref[...] = jnp.dot(x_ref[...], ws_ref[...],
                         preferred_element_type=jnp.float32) + b_ref[...]


def _convb1_body(a_ref, p_ref, wn_ref, o_ref):
    agg = p_ref[0] + p_ref[1]
    acc = a_ref[...] + jnp.dot(agg, wn_ref[...],
                               preferred_element_type=jnp.float32)
    o_ref[...] = jnp.maximum(acc, 0.0)


def _convb2_body(a_ref, p_ref, res_ref, wn_ref, o_ref):
    agg = p_ref[0] + p_ref[1]
    acc = a_ref[...] + res_ref[...] + jnp.dot(
        agg, wn_ref[...], preferred_element_type=jnp.float32)
    o_ref[...] = jnp.maximum(acc, 0.0)


_row_spec = pl.BlockSpec((_R, D), lambda i: (i, 0))
_p_spec = pl.BlockSpec((NC, _R, D), lambda i: (0, i, 0))
_w_spec = pl.BlockSpec((D, D), lambda i: (0, 0))
_b_spec = pl.BlockSpec((1, D), lambda i: (0, 0))
_out_shape = jax.ShapeDtypeStruct((N, D), jnp.float32)

_conva = pl.pallas_call(
    _conva_body,
    grid=(_GRID,),
    in_specs=[_row_spec, _w_spec, _b_spec],
    out_specs=_row_spec,
    out_shape=_out_shape,
)

_convb1 = pl.pallas_call(
    _convb1_body,
    grid=(_GRID,),
    in_specs=[_row_spec, _p_spec, _w_spec],
    out_specs=_row_spec,
    out_shape=_out_shape,
)

_convb2 = pl.pallas_call(
    _convb2_body,
    grid=(_GRID,),
    in_specs=[_row_spec, _p_spec, _row_spec, _w_spec],
    out_specs=_row_spec,
    out_shape=_out_shape,
)


def kernel(x, edge_index, Ws1_0, Wn1_0, b1_0, Ws2_0, Wn2_0, b2_0,
           Ws1_1, Wn1_1, b1_1, Ws2_1, Wn2_1, b2_1):
    src = edge_index[0]
    dst = edge_index[1]
    zeros = jnp.zeros((N_PAD, D), jnp.float32)

    def block(x, Ws1, Wn1, b1, Ws2, Wn2, b2):
        p1 = _spmm(x, src, dst, zeros)
        a1 = _conva(x, Ws1, b1.reshape(1, D))
        h = _convb1(a1, p1, Wn1)
        p2 = _spmm(h, src, dst, zeros)
        a2 = _conva(h, Ws2, b2.reshape(1, D))
        return _convb2(a2, p2, x, Wn2)

    x = block(x, Ws1_0, Wn1_0, b1_0, Ws2_0, Wn2_0, b2_0)
    x = block(x, Ws1_1, Wn1_1, b1_1, Ws2_1, Wn2_1, b2_1)
    return (x, edge_index)


# speedup vs baseline: 12.9765x; 1.0174x over previous
import functools

import jax
import jax.numpy as jnp
from jax import lax
from jax.experimental import pallas as pl
from jax.experimental.pallas import tpu as pltpu
from jax.experimental.pallas import tpu_sc as plsc

N = 10000
D = 128
E = 320000

NC = 2
NS = 16
NW = NC * NS
EPW = E // NW
CHUNK = 128
STEPS = EPW // CHUNK
TAIL = EPW - STEPS * CHUNK
N_PAD = 10112
RPT = N_PAD // NS


def _spmm_body(x_hbm, ei_hbm, zeros_hbm, out_hbm,
               src_v, dst0, dst1, dstt, rows0, rows1, rowst,
               acc_sh, g0, g1, d0, d1, s0, s1):
    c = lax.axis_index("c")
    s = lax.axis_index("s")
    wid = s * NC + c
    r0 = s * RPT
    eb = wid * EPW
    pltpu.sync_copy(ei_hbm.at[pl.ds(eb, EPW)], src_v)
    pltpu.async_copy(ei_hbm.at[pl.ds(E + eb, CHUNK)], dst0, d0)
    pltpu.async_copy(x_hbm.at[src_v.at[pl.ds(0, CHUNK)]], rows0, g0)
    pltpu.sync_copy(zeros_hbm.at[pl.ds(r0, RPT)], acc_sh.at[pl.ds(r0, RPT)])
    plsc.subcore_barrier()

    def halfstep(i, rows, dst, gsem, dsem, ssem,
                 orows, odst, ogsem, odsem, ossem):
        @pl.when(i >= 1)
        def _():
            pltpu.make_async_copy(orows, acc_sh.at[odst], ossem).wait()

        @pl.when(i + 1 < STEPS)
        def _():
            pltpu.async_copy(
                ei_hbm.at[pl.ds(E + eb + (i + 1) * CHUNK, CHUNK)], odst, odsem)
            pltpu.async_copy(
                x_hbm.at[src_v.at[pl.ds((i + 1) * CHUNK, CHUNK)]], orows, ogsem)

        pltpu.make_async_copy(
            ei_hbm.at[pl.ds(E + eb + i * CHUNK, CHUNK)], dst, dsem).wait()
        pltpu.make_async_copy(
            x_hbm.at[src_v.at[pl.ds(i * CHUNK, CHUNK)]], rows, gsem).wait()
        pltpu.async_copy(rows, acc_sh.at[dst], ssem, add=True)

    def step(i, carry):
        @pl.when(lax.rem(i, 2) == 0)
        def _():
            halfstep(i, rows0, dst0, g0, d0, s0, rows1, dst1, g1, d1, s1)

        @pl.when(lax.rem(i, 2) == 1)
        def _():
            halfstep(i, rows1, dst1, g1, d1, s1, rows0, dst0, g0, d0, s0)

        return carry

    lax.fori_loop(0, STEPS, step, 0)
    pltpu.make_async_copy(rows1, acc_sh.at[dst1], s1).wait()
    tb = eb + STEPS * CHUNK
    pltpu.sync_copy(ei_hbm.at[pl.ds(E + tb, TAIL)], dstt)
    pltpu.async_copy(x_hbm.at[src_v.at[pl.ds(STEPS * CHUNK, TAIL)]], rowst, g0)
    pltpu.make_async_copy(
        x_hbm.at[src_v.at[pl.ds(STEPS * CHUNK, TAIL)]], rowst, g0).wait()
    pltpu.sync_copy(rowst, acc_sh.at[dstt], add=True)
    plsc.subcore_barrier()
    pltpu.sync_copy(acc_sh.at[pl.ds(r0, RPT)], out_hbm.at[c, pl.ds(r0, RPT)])


def _make_spmm():
    mesh = plsc.VectorSubcoreMesh(core_axis_name="c", subcore_axis_name="s")
    return functools.partial(
        pl.kernel,
        mesh=mesh,
        out_type=jax.ShapeDtypeStruct((NC, N_PAD, D), jnp.float32),
        scratch_types=[
            pltpu.VMEM((EPW,), jnp.int32),
            pltpu.VMEM((CHUNK,), jnp.int32),
            pltpu.VMEM((CHUNK,), jnp.int32),
            pltpu.VMEM((TAIL,), jnp.int32),
            pltpu.VMEM((CHUNK, D), jnp.float32),
            pltpu.VMEM((CHUNK, D), jnp.float32),
            pltpu.VMEM((TAIL, D), jnp.float32),
            pltpu.VMEM_SHARED((N_PAD, D), jnp.float32),
            pltpu.SemaphoreType.DMA,
            pltpu.SemaphoreType.DMA,
            pltpu.SemaphoreType.DMA,
            pltpu.SemaphoreType.DMA,
            pltpu.SemaphoreType.DMA,
            pltpu.SemaphoreType.DMA,
        ],
    )(_spmm_body)


_spmm = _make_spmm()


_R = 2000
_GRID = N // _R


def _conva_body(x_ref, ws_ref, b_ref, o_ref):
    o_ref[...] = jnp.dot(x_ref[...], ws_ref[...],
                         preferred_element_type=jnp.float32) + b_ref[...]


def _convb1_body(a_ref, p_ref, wn_ref, o_ref):
    agg = p_ref[0] + p_ref[1]
    acc = a_ref[...] + jnp.dot(agg, wn_ref[...],
                               preferred_element_type=jnp.float32)
    o_ref[...] = jnp.maximum(acc, 0.0)


def _convb2_body(a_ref, p_ref, res_ref, wn_ref, o_ref):
    agg = p_ref[0] + p_ref[1]
    acc = a_ref[...] + res_ref[...] + jnp.dot(
        agg, wn_ref[...], preferred_element_type=jnp.float32)
    o_ref[...] = jnp.maximum(acc, 0.0)


_row_spec = pl.BlockSpec((_R, D), lambda i: (i, 0))
_p_spec = pl.BlockSpec((NC, _R, D), lambda i: (0, i, 0))
_w_spec = pl.BlockSpec((D, D), lambda i: (0, 0))
_b_spec = pl.BlockSpec((1, D), lambda i: (0, 0))
_out_shape = jax.ShapeDtypeStruct((N, D), jnp.float32)

_conva = pl.pallas_call(
    _conva_body,
    grid=(_GRID,),
    in_specs=[_row_spec, _w_spec, _b_spec],
    out_specs=_row_spec,
    out_shape=_out_shape,
)

_convb1 = pl.pallas_call(
    _convb1_body,
    grid=(_GRID,),
    in_specs=[_row_spec, _p_spec, _w_spec],
    out_specs=_row_spec,
    out_shape=_out_shape,
)

_convb2 = pl.pallas_call(
    _convb2_body,
    grid=(_GRID,),
    in_specs=[_row_spec, _p_spec, _row_spec, _w_spec],
    out_specs=_row_spec,
    out_shape=_out_shape,
)


def kernel(x, edge_index, Ws1_0, Wn1_0, b1_0, Ws2_0, Wn2_0, b2_0,
           Ws1_1, Wn1_1, b1_1, Ws2_1, Wn2_1, b2_1):
    zeros = jnp.zeros((N_PAD, D), jnp.float32)
    ei = edge_index.reshape(2 * E)

    def block(x, Ws1, Wn1, b1, Ws2, Wn2, b2):
        p1 = _spmm(x, ei, zeros)
        a1 = _conva(x, Ws1, b1.reshape(1, D))
        h = _convb1(a1, p1, Wn1)
        p2 = _spmm(h, ei, zeros)
        a2 = _conva(h, Ws2, b2.reshape(1, D))
        return _convb2(a2, p2, x, Wn2)

    x = block(x, Ws1_0, Wn1_0, b1_0, Ws2_0, Wn2_0, b2_0)
    x = block(x, Ws1_1, Wn1_1, b1_1, Ws2_1, Wn2_1, b2_1)
    return (x, edge_index)
